# baseline (speedup 1.0000x reference)
import jax
import jax.numpy as jnp
from jax.experimental import pallas as pl
from jax.experimental.pallas import tpu as pltpu

_DIM = 1000
_B = 64
_K = 4
_NS = 2
_CHUNK = _B // _K


def _onehot_body(idx_ref, out_hbm, scratch, sems):
    i = pl.program_id(0)
    ni = pl.num_programs(0)
    n1 = idx_ref.shape[1]
    slot = jax.lax.rem(i, _NS)

    def _copy(step, j):
        s = jax.lax.rem(step, _NS)
        base = step * _B + j * _CHUNK
        return pltpu.make_async_copy(
            scratch.at[s, pl.ds(j * _CHUNK, _CHUNK)],
            out_hbm.at[pl.ds(base, _CHUNK)],
            sems.at[s, j],
        )

    @pl.when(i >= _NS)
    def _wait_prev():
        for j in range(_K):
            _copy(i - _NS, j).wait()

    idx = idx_ref[...]
    iota = jax.lax.broadcasted_iota(jnp.int32, (_B, n1, _DIM), 2)
    scratch[slot] = (iota == idx[:, :, None]).astype(jnp.float32)

    for j in range(_K):
        _copy(i, j).start(priority=j % 2)

    @pl.when(i == ni - 1)
    def _drain():
        for step_back in range(_NS - 1, -1, -1):
            for j in range(_K):
                _copy(i - step_back, j).wait()


def kernel(tensor):
    n0, n1 = tensor.shape
    idx = tensor.astype(jnp.int32)
    return pl.pallas_call(
        _onehot_body,
        grid=(n0 // _B,),
        in_specs=[pl.BlockSpec((_B, n1), lambda i: (i, 0))],
        out_specs=pl.BlockSpec(memory_space=pl.ANY),
        out_shape=jax.ShapeDtypeStruct((n0, n1, _DIM), jnp.float32),
        scratch_shapes=[
            pltpu.VMEM((_NS, _B, n1, _DIM), jnp.float32),
            pltpu.SemaphoreType.DMA((_NS, _K)),
        ],
    )(idx)


# speedup vs baseline: 4.7769x; 4.7769x over previous
import jax
import jax.numpy as jnp
from jax.experimental import pallas as pl

_DIM = 1000
_R = 2048


def _onehot_body(idx_ref, out_ref):
    idx = idx_ref[...]
    iota = jax.lax.broadcasted_iota(jnp.int32, (1, _DIM, _R), 1)
    out_ref[...] = (iota == idx).astype(jnp.float32)


def kernel(tensor):
    n0, n1 = tensor.shape
    idx_t = tensor.astype(jnp.int32).T.reshape(n1, 1, n0)
    out_phys = pl.pallas_call(
        _onehot_body,
        grid=(n1, n0 // _R),
        in_specs=[pl.BlockSpec((1, 1, _R), lambda c, r: (c, 0, r))],
        out_specs=pl.BlockSpec((1, _DIM, _R), lambda c, r: (c, 0, r)),
        out_shape=jax.ShapeDtypeStruct((n1, _DIM, n0), jnp.float32),
    )(idx_t)
    return jnp.transpose(out_phys, (2, 0, 1))
